# baseline (speedup 1.0000x reference)
import functools

import jax
import jax.numpy as jnp
import numpy as np
from jax import lax
from jax.experimental import pallas as pl
from jax.experimental.pallas import tpu as pltpu
from jax.experimental.pallas import tpu_sc as plsc

N = 10000
E = 160000
MUL0 = 16
MUL1 = 4
DIM = MUL0 + 3 * MUL1
RMH = 64
WEXP = 480
LDIM = 44
ODIM = 44

NW = 32
CHUNK = 128
NCHUNK = 40
EP = NW * NCHUNK * CHUNK
ROWS_PER_TILE = N // 16

BLK_E = 2048


FDIM = 112


def _build_consts():
    c_w1 = 1.0 / (4.0 * np.sqrt(2.0))
    c_w2 = 1.0 / (np.sqrt(3.0) * 2.0 * np.sqrt(2.0))
    c_w3 = 1.0 / (4.0 * np.sqrt(3.0))
    c_w4 = 1.0 / (2.0 * np.sqrt(3.0))
    c_w5 = 1.0 / (2.0 * np.sqrt(6.0))
    Rtile = np.zeros((32, FDIM), np.float32)
    Rrep = np.zeros((4, FDIM), np.float32)
    for kp in range(4):
        for c in range(DIM):
            Rtile[c, 28 * kp + c] = 1.0
            Rrep[kp, 28 * kp + c] = 1.0
    R112 = np.zeros((FDIM, WEXP), np.float32)
    BigS = np.zeros((WEXP, 32), np.float32)
    wscale = np.zeros((WEXP,), np.float32)
    wscale[0:256] = c_w1
    wscale[256:320] = c_w2
    wscale[320:384] = c_w3
    wscale[384:432] = c_w4
    wscale[432:480] = c_w5
    for i in range(16):
        for o in range(16):
            R112[i, 16 * i + o] = 1.0
            BigS[16 * i + o, o] = 1.0
        for v in range(4):
            R112[i, 320 + 4 * i + v] = 1.0
            BigS[320 + 4 * i + v, 28 + v] = 1.0
    for u in range(4):
        for o in range(16):
            for k in range(3):
                R112[28 * (k + 1) + 16 + 3 * u + k, 256 + 16 * u + o] = 1.0
            BigS[256 + 16 * u + o, o] = 1.0
    for k in range(3):
        a, b = (k + 1) % 3, (k + 2) % 3
        for u in range(4):
            for v in range(4):
                R112[16 + 3 * u + k, 384 + 16 * k + 4 * u + v] = 1.0
                BigS[384 + 16 * k + 4 * u + v, 16 + 3 * v + k] = 1.0
                R112[28 * (b + 1) + 16 + 3 * u + a, 432 + 16 * k + 4 * u + v] = 1.0
                R112[28 * (a + 1) + 16 + 3 * u + b, 432 + 16 * k + 4 * u + v] = -1.0
                BigS[432 + 16 * k + 4 * u + v, 16 + 3 * v + k] = 1.0
    Rsv32 = np.zeros((32, 32), np.float32)
    Ryk32 = np.zeros((4, 32), np.float32)
    for v in range(4):
        for k in range(3):
            Rsv32[28 + v, 16 + 3 * v + k] = 1.0
            Ryk32[1 + k, 16 + 3 * v + k] = 1.0
    mask32 = np.ones((1, 32), np.float32)
    mask32[0, 28:] = 0.0
    ones31 = np.ones((3, 1), np.float32)
    return Rtile, Rrep, R112, BigS, Rsv32, Ryk32, mask32, ones31, wscale


(_Rtile, _Rrep, _R112, _BigS, _Rsv32, _Ryk32, _mask32, _ones31,
 _wscale) = _build_consts()

def _sc_gather_body(h_hbm, idx_hbm, out_hbm, idx_v, buf0, buf1, sem0, sem1):
    wid = lax.axis_index("s") * 2 + lax.axis_index("c")
    base = wid * (NCHUNK * CHUNK)
    pltpu.sync_copy(idx_hbm.at[wid], idx_v)
    bufs = (buf0, buf1)
    sems = (sem0, sem1)
    pltpu.async_copy(h_hbm.at[idx_v.at[0]], buf0, sem0)

    def body(j, _):
        slot = lax.rem(j, 2)

        @pl.when(j + 1 < NCHUNK)
        def _():
            for s in range(2):
                @pl.when(slot != s)
                def _():
                    pltpu.async_copy(h_hbm.at[idx_v.at[j + 1]], bufs[s], sems[s])

        for s in range(2):
            @pl.when(slot == s)
            def _():
                pltpu.make_async_copy(h_hbm.at[idx_v.at[j]], bufs[s], sems[s]).wait()
                pltpu.sync_copy(bufs[s], out_hbm.at[pl.ds(base + j * CHUNK, CHUNK)])
        return 0

    lax.fori_loop(0, NCHUNK, body, 0)


def _sc_scatter_body(msg_hbm, idx_hbm, zero_hbm, out_hbm,
                     idx_v, buf0, buf1, agg_sh, sem0, sem1):
    cid = lax.axis_index("c")
    sid = lax.axis_index("s")
    wid = sid * 2 + cid
    base = wid * (NCHUNK * CHUNK)
    pltpu.sync_copy(zero_hbm.at[pl.ds(sid * ROWS_PER_TILE, ROWS_PER_TILE)],
                    agg_sh.at[pl.ds(sid * ROWS_PER_TILE, ROWS_PER_TILE)])
    pltpu.sync_copy(idx_hbm.at[wid], idx_v)
    plsc.subcore_barrier()
    bufs = (buf0, buf1)
    sems = (sem0, sem1)
    pltpu.async_copy(msg_hbm.at[pl.ds(base, CHUNK)], buf0, sem0)

    def body(j, _):
        slot = lax.rem(j, 2)

        @pl.when(j + 1 < NCHUNK)
        def _():
            for s in range(2):
                @pl.when(slot != s)
                def _():
                    pltpu.async_copy(
                        msg_hbm.at[pl.ds(base + (j + 1) * CHUNK, CHUNK)],
                        bufs[s], sems[s])

        for s in range(2):
            @pl.when(slot == s)
            def _():
                pltpu.make_async_copy(
                    msg_hbm.at[pl.ds(base + j * CHUNK, CHUNK)],
                    bufs[s], sems[s]).wait()
                pltpu.sync_copy(bufs[s], agg_sh.at[idx_v.at[j]], add=True)
        return 0

    lax.fori_loop(0, NCHUNK, body, 0)
    plsc.subcore_barrier()
    pltpu.sync_copy(agg_sh.at[pl.ds(sid * ROWS_PER_TILE, ROWS_PER_TILE)],
                    out_hbm.at[cid, pl.ds(sid * ROWS_PER_TILE, ROWS_PER_TILE)])


@functools.cache
def _make_sc_kernels():
    mesh = plsc.VectorSubcoreMesh(core_axis_name="c", subcore_axis_name="s")
    params = pltpu.CompilerParams(use_tc_tiling_on_sc=False)
    gather = pl.kernel(
        _sc_gather_body,
        out_type=jax.ShapeDtypeStruct((EP, 32), jnp.float32),
        mesh=mesh,
        compiler_params=params,
        scratch_types=[
            pltpu.VMEM((NCHUNK, CHUNK), jnp.int32),
            pltpu.VMEM((CHUNK, 32), jnp.float32),
            pltpu.VMEM((CHUNK, 32), jnp.float32),
            pltpu.SemaphoreType.DMA,
            pltpu.SemaphoreType.DMA,
        ],
    )
    scatter = pl.kernel(
        _sc_scatter_body,
        out_type=jax.ShapeDtypeStruct((2, N, 32), jnp.float32),
        mesh=mesh,
        compiler_params=params,
        scratch_types=[
            pltpu.VMEM((NCHUNK, CHUNK), jnp.int32),
            pltpu.VMEM((CHUNK, 32), jnp.float32),
            pltpu.VMEM((CHUNK, 32), jnp.float32),
            pltpu.VMEM_SHARED((N, 32), jnp.float32),
            pltpu.SemaphoreType.DMA,
            pltpu.SemaphoreType.DMA,
        ],
    )
    return gather, scatter


def _tp_body(ev_ref, elen_ref, xg_ref, W1_ref, b1_ref, W2_ref, b2_ref,
             Rtile_ref, Rrep_ref, R112_ref, BigS_ref, Rsv_ref, Ryk_ref,
             mask_ref, ones31_ref, out_ref):
    f32 = jnp.float32
    ev = ev_ref[...]
    elen = elen_ref[...]
    xg = xg_ref[...]
    r2 = jnp.dot(ev * ev, ones31_ref[...], preferred_element_type=f32)
    r = jnp.sqrt(r2)
    y1 = ev * (np.float32(np.sqrt(3.0)) / jnp.maximum(r, 1e-12))
    y1e = jnp.concatenate([jnp.ones_like(r), y1], axis=1)
    pre = elen * W1_ref[...] + b1_ref[...]
    hid = pre * jax.nn.sigmoid(pre)
    wexp = (jnp.dot(hid.astype(jnp.bfloat16), W2_ref[...],
                    preferred_element_type=f32) + b2_ref[...])
    xgrep = jnp.dot(xg, Rtile_ref[...], preferred_element_type=f32)
    Y = jnp.dot(y1e, Rrep_ref[...], preferred_element_type=f32)
    F = xgrep * Y
    Lexp = jnp.dot(F.astype(jnp.bfloat16), R112_ref[...],
                   preferred_element_type=f32)
    OUT = jnp.dot((Lexp * wexp).astype(jnp.bfloat16), BigS_ref[...],
                  preferred_element_type=f32)
    souter = (jnp.dot(OUT, Rsv_ref[...], preferred_element_type=f32)
              * jnp.dot(y1e, Ryk_ref[...], preferred_element_type=f32))
    msg = OUT * mask_ref[...] + souter
    row = (pl.program_id(0) * BLK_E
           + lax.broadcasted_iota(jnp.int32, (BLK_E, 1), 0))
    out_ref[...] = jnp.where(row < E, msg, 0.0)


def _gate_body(h_ref, part_ref, Wg_ref, bg_ref, out_ref):
    h = h_ref[...]
    agg = part_ref[0] + part_ref[1]
    gate = jax.nn.sigmoid(
        jnp.dot(h[:, :MUL0], Wg_ref[...], preferred_element_type=jnp.float32)
        + bg_ref[...])
    out_ref[...] = h + jnp.concatenate(
        [agg[:, :MUL0], agg[:, MUL0:DIM] * gate], axis=1)


def _full(shape):
    return pl.BlockSpec(shape, lambda i: tuple(0 for _ in shape))


@jax.jit
def kernel(h, edge_index, edge_vec, edge_len, W1, b1, W2, b2, Wg, bg):
    sender = edge_index[0]
    receiver = edge_index[1]
    sp = jnp.pad(sender, (0, EP - E)).reshape(NW, NCHUNK, CHUNK)
    rp = jnp.pad(receiver, (0, EP - E)).reshape(NW, NCHUNK, CHUNK)
    h32 = jnp.pad(h, ((0, 0), (0, 32 - DIM)))
    evp = jnp.pad(edge_vec, ((0, EP - E), (0, 0)))
    elp = jnp.pad(edge_len, (0, EP - E)).reshape(EP, 1)
    zero_init = jnp.zeros((N, 32), jnp.float32)
    wsc = jnp.asarray(_wscale)
    W2e = jnp.concatenate([W2[:, :384], W2[:, 384:400], W2[:, 384:400],
                           W2[:, 384:400], W2[:, 400:416], W2[:, 400:416],
                           W2[:, 400:416]], axis=1) * wsc
    b2e = jnp.concatenate([b2[:384], b2[384:400], b2[384:400], b2[384:400],
                           b2[400:416], b2[400:416], b2[400:416]]) * wsc
    W2eb = W2e.astype(jnp.bfloat16)

    sc_gather, sc_scatter = _make_sc_kernels()
    xg = sc_gather(h32, sp)

    grid = EP // BLK_E
    msg = pl.pallas_call(
        _tp_body,
        grid=(grid,),
        in_specs=[
            pl.BlockSpec((BLK_E, 3), lambda i: (i, 0)),
            pl.BlockSpec((BLK_E, 1), lambda i: (i, 0)),
            pl.BlockSpec((BLK_E, 32), lambda i: (i, 0)),
            _full((1, RMH)), _full((1, RMH)), _full((RMH, WEXP)),
            _full((1, WEXP)), _full((32, FDIM)), _full((4, FDIM)),
            _full((FDIM, WEXP)), _full((WEXP, 32)), _full((32, 32)),
            _full((4, 32)), _full((1, 32)), _full((3, 1)),
        ],
        out_specs=pl.BlockSpec((BLK_E, 32), lambda i: (i, 0)),
        out_shape=jax.ShapeDtypeStruct((EP, 32), jnp.float32),
    )(evp, elp, xg,
      W1, b1.reshape(1, RMH), W2eb, b2e.reshape(1, WEXP),
      jnp.asarray(_Rtile), jnp.asarray(_Rrep),
      jnp.asarray(_R112).astype(jnp.bfloat16),
      jnp.asarray(_BigS).astype(jnp.bfloat16),
      jnp.asarray(_Rsv32), jnp.asarray(_Ryk32),
      jnp.asarray(_mask32), jnp.asarray(_ones31))

    parts = sc_scatter(msg, rp, zero_init)

    out = pl.pallas_call(
        _gate_body,
        grid=(1,),
        in_specs=[
            _full((N, DIM)), _full((2, N, 32)),
            _full((MUL0, DIM - MUL0)), _full((1, DIM - MUL0)),
        ],
        out_specs=_full((N, DIM)),
        out_shape=jax.ShapeDtypeStruct((N, DIM), jnp.float32),
    )(h, parts, Wg, bg.reshape(1, DIM - MUL0))
    return out


# speedup vs baseline: 1.0099x; 1.0099x over previous
import functools

import jax
import jax.numpy as jnp
import numpy as np
from jax import lax
from jax.experimental import pallas as pl
from jax.experimental.pallas import tpu as pltpu
from jax.experimental.pallas import tpu_sc as plsc

N = 10000
E = 160000
MUL0 = 16
MUL1 = 4
DIM = MUL0 + 3 * MUL1
RMH = 64
WEXP = 480
LDIM = 44
ODIM = 44

NW = 32
CHUNK = 128
NCHUNK = 40
EP = NW * NCHUNK * CHUNK
ROWS_PER_TILE = N // 16

BLK_E = 2048


FDIM = 112


def _build_consts():
    c_w1 = 1.0 / (4.0 * np.sqrt(2.0))
    c_w2 = 1.0 / (np.sqrt(3.0) * 2.0 * np.sqrt(2.0))
    c_w3 = 1.0 / (4.0 * np.sqrt(3.0))
    c_w4 = 1.0 / (2.0 * np.sqrt(3.0))
    c_w5 = 1.0 / (2.0 * np.sqrt(6.0))
    Rtile = np.zeros((32, FDIM), np.float32)
    Rrep = np.zeros((4, FDIM), np.float32)
    for kp in range(4):
        for c in range(DIM):
            Rtile[c, 28 * kp + c] = 1.0
            Rrep[kp, 28 * kp + c] = 1.0
    R112 = np.zeros((FDIM, WEXP), np.float32)
    BigS = np.zeros((WEXP, 32), np.float32)
    wscale = np.zeros((WEXP,), np.float32)
    wscale[0:256] = c_w1
    wscale[256:320] = c_w2
    wscale[320:384] = c_w3
    wscale[384:432] = c_w4
    wscale[432:480] = c_w5
    for i in range(16):
        for o in range(16):
            R112[i, 16 * i + o] = 1.0
            BigS[16 * i + o, o] = 1.0
        for v in range(4):
            R112[i, 320 + 4 * i + v] = 1.0
            BigS[320 + 4 * i + v, 28 + v] = 1.0
    for u in range(4):
        for o in range(16):
            for k in range(3):
                R112[28 * (k + 1) + 16 + 3 * u + k, 256 + 16 * u + o] = 1.0
            BigS[256 + 16 * u + o, o] = 1.0
    for k in range(3):
        a, b = (k + 1) % 3, (k + 2) % 3
        for u in range(4):
            for v in range(4):
                R112[16 + 3 * u + k, 384 + 16 * k + 4 * u + v] = 1.0
                BigS[384 + 16 * k + 4 * u + v, 16 + 3 * v + k] = 1.0
                R112[28 * (b + 1) + 16 + 3 * u + a, 432 + 16 * k + 4 * u + v] = 1.0
                R112[28 * (a + 1) + 16 + 3 * u + b, 432 + 16 * k + 4 * u + v] = -1.0
                BigS[432 + 16 * k + 4 * u + v, 16 + 3 * v + k] = 1.0
    Rsv32 = np.zeros((32, 32), np.float32)
    Ryk32 = np.zeros((4, 32), np.float32)
    for v in range(4):
        for k in range(3):
            Rsv32[28 + v, 16 + 3 * v + k] = 1.0
            Ryk32[1 + k, 16 + 3 * v + k] = 1.0
    mask32 = np.ones((1, 32), np.float32)
    mask32[0, 28:] = 0.0
    ones31 = np.ones((3, 1), np.float32)
    return Rtile, Rrep, R112, BigS, Rsv32, Ryk32, mask32, ones31, wscale


(_Rtile, _Rrep, _R112, _BigS, _Rsv32, _Ryk32, _mask32, _ones31,
 _wscale) = _build_consts()

NBUF = 4


def _sc_gather_body(h_hbm, idx_hbm, out_hbm, idx_v,
                    buf0, buf1, buf2, buf3,
                    gs0, gs1, gs2, gs3, ws0, ws1, ws2, ws3):
    wid = lax.axis_index("s") * 2 + lax.axis_index("c")
    base = wid * (NCHUNK * CHUNK)
    pltpu.sync_copy(idx_hbm.at[wid], idx_v)
    bufs = (buf0, buf1, buf2, buf3)
    gsems = (gs0, gs1, gs2, gs3)
    wsems = (ws0, ws1, ws2, ws3)
    for b in range(NBUF):
        pltpu.async_copy(h_hbm.at[idx_v.at[b]], bufs[b], gsems[b])

    def body(jo, _):
        for b in range(NBUF):
            j = jo * NBUF + b
            pltpu.make_async_copy(
                h_hbm.at[idx_v.at[j]], bufs[b], gsems[b]).wait()
            pltpu.async_copy(
                bufs[b], out_hbm.at[pl.ds(base + j * CHUNK, CHUNK)], wsems[b])

            @pl.when(j + NBUF < NCHUNK)
            def _():
                pltpu.make_async_copy(
                    bufs[b], out_hbm.at[pl.ds(base, CHUNK)], wsems[b]).wait()
                pltpu.async_copy(h_hbm.at[idx_v.at[j + NBUF]], bufs[b],
                                 gsems[b])
        return 0

    lax.fori_loop(0, NCHUNK // NBUF, body, 0)
    for b in range(NBUF):
        pltpu.make_async_copy(
            bufs[b], out_hbm.at[pl.ds(base, CHUNK)], wsems[b]).wait()


def _sc_scatter_body(msg_hbm, idx_hbm, zero_hbm, out_hbm,
                     idx_v, buf0, buf1, agg_sh, sem0, sem1):
    cid = lax.axis_index("c")
    sid = lax.axis_index("s")
    wid = sid * 2 + cid
    base = wid * (NCHUNK * CHUNK)
    pltpu.sync_copy(zero_hbm.at[pl.ds(sid * ROWS_PER_TILE, ROWS_PER_TILE)],
                    agg_sh.at[pl.ds(sid * ROWS_PER_TILE, ROWS_PER_TILE)])
    pltpu.sync_copy(idx_hbm.at[wid], idx_v)
    plsc.subcore_barrier()
    bufs = (buf0, buf1)
    sems = (sem0, sem1)
    pltpu.async_copy(msg_hbm.at[pl.ds(base, CHUNK)], buf0, sem0)

    def body(j, _):
        slot = lax.rem(j, 2)

        @pl.when(j + 1 < NCHUNK)
        def _():
            for s in range(2):
                @pl.when(slot != s)
                def _():
                    pltpu.async_copy(
                        msg_hbm.at[pl.ds(base + (j + 1) * CHUNK, CHUNK)],
                        bufs[s], sems[s])

        for s in range(2):
            @pl.when(slot == s)
            def _():
                pltpu.make_async_copy(
                    msg_hbm.at[pl.ds(base + j * CHUNK, CHUNK)],
                    bufs[s], sems[s]).wait()
                pltpu.sync_copy(bufs[s], agg_sh.at[idx_v.at[j]], add=True)
        return 0

    lax.fori_loop(0, NCHUNK, body, 0)
    plsc.subcore_barrier()
    pltpu.sync_copy(agg_sh.at[pl.ds(sid * ROWS_PER_TILE, ROWS_PER_TILE)],
                    out_hbm.at[cid, pl.ds(sid * ROWS_PER_TILE, ROWS_PER_TILE)])


@functools.cache
def _make_sc_kernels():
    mesh = plsc.VectorSubcoreMesh(core_axis_name="c", subcore_axis_name="s")
    params = pltpu.CompilerParams(use_tc_tiling_on_sc=False)
    gather = pl.kernel(
        _sc_gather_body,
        out_type=jax.ShapeDtypeStruct((EP, 32), jnp.float32),
        mesh=mesh,
        compiler_params=params,
        scratch_types=(
            [pltpu.VMEM((NCHUNK, CHUNK), jnp.int32)]
            + [pltpu.VMEM((CHUNK, 32), jnp.float32)] * NBUF
            + [pltpu.SemaphoreType.DMA] * (2 * NBUF)
        ),
    )
    scatter = pl.kernel(
        _sc_scatter_body,
        out_type=jax.ShapeDtypeStruct((2, N, 32), jnp.float32),
        mesh=mesh,
        compiler_params=params,
        scratch_types=[
            pltpu.VMEM((NCHUNK, CHUNK), jnp.int32),
            pltpu.VMEM((CHUNK, 32), jnp.float32),
            pltpu.VMEM((CHUNK, 32), jnp.float32),
            pltpu.VMEM_SHARED((N, 32), jnp.float32),
            pltpu.SemaphoreType.DMA,
            pltpu.SemaphoreType.DMA,
        ],
    )
    return gather, scatter


def _tp_body(ev_ref, elen_ref, xg_ref, W1_ref, b1_ref, W2_ref, b2_ref,
             Rtile_ref, Rrep_ref, R112_ref, BigS_ref, Rsv_ref, Ryk_ref,
             mask_ref, ones31_ref, out_ref):
    f32 = jnp.float32
    ev = ev_ref[...]
    elen = elen_ref[...]
    xg = xg_ref[...]
    r2 = jnp.dot(ev * ev, ones31_ref[...], preferred_element_type=f32)
    r = jnp.sqrt(r2)
    y1 = ev * (np.float32(np.sqrt(3.0)) / jnp.maximum(r, 1e-12))
    y1e = jnp.concatenate([jnp.ones_like(r), y1], axis=1)
    pre = elen * W1_ref[...] + b1_ref[...]
    hid = pre * jax.nn.sigmoid(pre)
    wexp = (jnp.dot(hid.astype(jnp.bfloat16), W2_ref[...],
                    preferred_element_type=f32) + b2_ref[...])
    xgrep = jnp.dot(xg, Rtile_ref[...], preferred_element_type=f32)
    Y = jnp.dot(y1e, Rrep_ref[...], preferred_element_type=f32)
    F = xgrep * Y
    Lexp = jnp.dot(F.astype(jnp.bfloat16), R112_ref[...],
                   preferred_element_type=f32)
    OUT = jnp.dot((Lexp * wexp).astype(jnp.bfloat16), BigS_ref[...],
                  preferred_element_type=f32)
    souter = (jnp.dot(OUT, Rsv_ref[...], preferred_element_type=f32)
              * jnp.dot(y1e, Ryk_ref[...], preferred_element_type=f32))
    msg = OUT * mask_ref[...] + souter
    row = (pl.program_id(0) * BLK_E
           + lax.broadcasted_iota(jnp.int32, (BLK_E, 1), 0))
    out_ref[...] = jnp.where(row < E, msg, 0.0)


def _gate_body(h_ref, part_ref, Wg_ref, bg_ref, out_ref):
    h = h_ref[...]
    agg = part_ref[0] + part_ref[1]
    gate = jax.nn.sigmoid(
        jnp.dot(h[:, :MUL0], Wg_ref[...], preferred_element_type=jnp.float32)
        + bg_ref[...])
    out_ref[...] = h + jnp.concatenate(
        [agg[:, :MUL0], agg[:, MUL0:DIM] * gate], axis=1)


def _full(shape):
    return pl.BlockSpec(shape, lambda i: tuple(0 for _ in shape))


@jax.jit
def kernel(h, edge_index, edge_vec, edge_len, W1, b1, W2, b2, Wg, bg):
    sender = edge_index[0]
    receiver = edge_index[1]
    sp = jnp.pad(sender, (0, EP - E)).reshape(NW, NCHUNK, CHUNK)
    rp = jnp.pad(receiver, (0, EP - E)).reshape(NW, NCHUNK, CHUNK)
    h32 = jnp.pad(h, ((0, 0), (0, 32 - DIM)))
    evp = jnp.pad(edge_vec, ((0, EP - E), (0, 0)))
    elp = jnp.pad(edge_len, (0, EP - E)).reshape(EP, 1)
    zero_init = jnp.zeros((N, 32), jnp.float32)
    wsc = jnp.asarray(_wscale)
    W2e = jnp.concatenate([W2[:, :384], W2[:, 384:400], W2[:, 384:400],
                           W2[:, 384:400], W2[:, 400:416], W2[:, 400:416],
                           W2[:, 400:416]], axis=1) * wsc
    b2e = jnp.concatenate([b2[:384], b2[384:400], b2[384:400], b2[384:400],
                           b2[400:416], b2[400:416], b2[400:416]]) * wsc
    W2eb = W2e.astype(jnp.bfloat16)

    sc_gather, sc_scatter = _make_sc_kernels()
    xg = sc_gather(h32, sp)

    grid = EP // BLK_E
    msg = pl.pallas_call(
        _tp_body,
        grid=(grid,),
        in_specs=[
            pl.BlockSpec((BLK_E, 3), lambda i: (i, 0)),
            pl.BlockSpec((BLK_E, 1), lambda i: (i, 0)),
            pl.BlockSpec((BLK_E, 32), lambda i: (i, 0)),
            _full((1, RMH)), _full((1, RMH)), _full((RMH, WEXP)),
            _full((1, WEXP)), _full((32, FDIM)), _full((4, FDIM)),
            _full((FDIM, WEXP)), _full((WEXP, 32)), _full((32, 32)),
            _full((4, 32)), _full((1, 32)), _full((3, 1)),
        ],
        out_specs=pl.BlockSpec((BLK_E, 32), lambda i: (i, 0)),
        out_shape=jax.ShapeDtypeStruct((EP, 32), jnp.float32),
    )(evp, elp, xg,
      W1, b1.reshape(1, RMH), W2eb, b2e.reshape(1, WEXP),
      jnp.asarray(_Rtile), jnp.asarray(_Rrep),
      jnp.asarray(_R112).astype(jnp.bfloat16),
      jnp.asarray(_BigS).astype(jnp.bfloat16),
      jnp.asarray(_Rsv32), jnp.asarray(_Ryk32),
      jnp.asarray(_mask32), jnp.asarray(_ones31))

    parts = sc_scatter(msg, rp, zero_init)

    out = pl.pallas_call(
        _gate_body,
        grid=(1,),
        in_specs=[
            _full((N, DIM)), _full((2, N, 32)),
            _full((MUL0, DIM - MUL0)), _full((1, DIM - MUL0)),
        ],
        out_specs=_full((N, DIM)),
        out_shape=jax.ShapeDtypeStruct((N, DIM), jnp.float32),
    )(h, parts, Wg, bg.reshape(1, DIM - MUL0))
    return out


# speedup vs baseline: 1.3706x; 1.3571x over previous
import functools

import jax
import jax.numpy as jnp
import numpy as np
from jax import lax
from jax.experimental import pallas as pl
from jax.experimental.pallas import tpu as pltpu
from jax.experimental.pallas import tpu_sc as plsc

N = 10000
E = 160000
MUL0 = 16
MUL1 = 4
DIM = MUL0 + 3 * MUL1
RMH = 64
WEXP = 480
LDIM = 44
ODIM = 44

NW = 32
CHUNK = 125
NCHUNK = 40
ROWS_PER_TILE = N // 16

BLK_E = 3200


FDIM = 112


def _build_consts():
    c_w1 = 1.0 / (4.0 * np.sqrt(2.0))
    c_w2 = 1.0 / (np.sqrt(3.0) * 2.0 * np.sqrt(2.0))
    c_w3 = 1.0 / (4.0 * np.sqrt(3.0))
    c_w4 = 1.0 / (2.0 * np.sqrt(3.0))
    c_w5 = 1.0 / (2.0 * np.sqrt(6.0))
    Rtile = np.zeros((32, FDIM), np.float32)
    Rrep = np.zeros((4, FDIM), np.float32)
    for kp in range(4):
        for c in range(DIM):
            Rtile[c, 28 * kp + c] = 1.0
            Rrep[kp, 28 * kp + c] = 1.0
    R112 = np.zeros((FDIM, WEXP), np.float32)
    BigS = np.zeros((WEXP, 32), np.float32)
    wscale = np.zeros((WEXP,), np.float32)
    wscale[0:256] = c_w1
    wscale[256:320] = c_w2
    wscale[320:384] = c_w3
    wscale[384:432] = c_w4
    wscale[432:480] = c_w5
    for i in range(16):
        for o in range(16):
            R112[i, 16 * i + o] = 1.0
            BigS[16 * i + o, o] = 1.0
        for v in range(4):
            R112[i, 320 + 4 * i + v] = 1.0
            BigS[320 + 4 * i + v, 28 + v] = 1.0
    for u in range(4):
        for o in range(16):
            for k in range(3):
                R112[28 * (k + 1) + 16 + 3 * u + k, 256 + 16 * u + o] = 1.0
            BigS[256 + 16 * u + o, o] = 1.0
    for k in range(3):
        a, b = (k + 1) % 3, (k + 2) % 3
        for u in range(4):
            for v in range(4):
                R112[16 + 3 * u + k, 384 + 16 * k + 4 * u + v] = 1.0
                BigS[384 + 16 * k + 4 * u + v, 16 + 3 * v + k] = 1.0
                R112[28 * (b + 1) + 16 + 3 * u + a, 432 + 16 * k + 4 * u + v] = 1.0
                R112[28 * (a + 1) + 16 + 3 * u + b, 432 + 16 * k + 4 * u + v] = -1.0
                BigS[432 + 16 * k + 4 * u + v, 16 + 3 * v + k] = 1.0
    Rsv32 = np.zeros((32, 32), np.float32)
    Ryk32 = np.zeros((4, 32), np.float32)
    for v in range(4):
        for k in range(3):
            Rsv32[28 + v, 16 + 3 * v + k] = 1.0
            Ryk32[1 + k, 16 + 3 * v + k] = 1.0
    mask32 = np.ones((1, 32), np.float32)
    mask32[0, 28:] = 0.0
    return Rtile, Rrep, R112, BigS, Rsv32, Ryk32, mask32, wscale


(_Rtile, _Rrep, _R112, _BigS, _Rsv32, _Ryk32, _mask32,
 _wscale) = _build_consts()

NBUF = 4


def _sc_gather_body(h_hbm, idx_hbm, out_hbm, idx_v,
                    buf0, buf1, buf2, buf3,
                    gs0, gs1, gs2, gs3, ws0, ws1, ws2, ws3):
    wid = lax.axis_index("s") * 2 + lax.axis_index("c")
    base = wid * (NCHUNK * CHUNK)
    pltpu.sync_copy(idx_hbm.at[wid], idx_v)
    bufs = (buf0, buf1, buf2, buf3)
    gsems = (gs0, gs1, gs2, gs3)
    wsems = (ws0, ws1, ws2, ws3)
    for b in range(NBUF):
        pltpu.async_copy(h_hbm.at[idx_v.at[b]], bufs[b], gsems[b])

    def body(jo, _):
        for b in range(NBUF):
            j = jo * NBUF + b
            pltpu.make_async_copy(
                h_hbm.at[idx_v.at[j]], bufs[b], gsems[b]).wait()
            pltpu.async_copy(
                bufs[b], out_hbm.at[pl.ds(base + j * CHUNK, CHUNK)], wsems[b])

            @pl.when(j + NBUF < NCHUNK)
            def _():
                pltpu.make_async_copy(
                    bufs[b], out_hbm.at[pl.ds(base, CHUNK)], wsems[b]).wait()
                pltpu.async_copy(h_hbm.at[idx_v.at[j + NBUF]], bufs[b],
                                 gsems[b])
        return 0

    lax.fori_loop(0, NCHUNK // NBUF, body, 0)
    for b in range(NBUF):
        pltpu.make_async_copy(
            bufs[b], out_hbm.at[pl.ds(base, CHUNK)], wsems[b]).wait()


def _sc_scatter_body(msg_hbm, idx_hbm, zero_hbm, out_hbm,
                     idx_v, buf0, buf1, agg_sh, sem0, sem1):
    cid = lax.axis_index("c")
    sid = lax.axis_index("s")
    wid = sid * 2 + cid
    base = wid * (NCHUNK * CHUNK)
    pltpu.sync_copy(zero_hbm.at[pl.ds(sid * ROWS_PER_TILE, ROWS_PER_TILE)],
                    agg_sh.at[pl.ds(sid * ROWS_PER_TILE, ROWS_PER_TILE)])
    pltpu.sync_copy(idx_hbm.at[wid], idx_v)
    plsc.subcore_barrier()
    bufs = (buf0, buf1)
    sems = (sem0, sem1)
    pltpu.async_copy(msg_hbm.at[pl.ds(base, CHUNK)], buf0, sem0)

    def body(j, _):
        slot = lax.rem(j, 2)

        @pl.when(j + 1 < NCHUNK)
        def _():
            for s in range(2):
                @pl.when(slot != s)
                def _():
                    pltpu.async_copy(
                        msg_hbm.at[pl.ds(base + (j + 1) * CHUNK, CHUNK)],
                        bufs[s], sems[s])

        for s in range(2):
            @pl.when(slot == s)
            def _():
                pltpu.make_async_copy(
                    msg_hbm.at[pl.ds(base + j * CHUNK, CHUNK)],
                    bufs[s], sems[s]).wait()
                pltpu.sync_copy(bufs[s], agg_sh.at[idx_v.at[j]], add=True)
        return 0

    lax.fori_loop(0, NCHUNK, body, 0)
    plsc.subcore_barrier()
    pltpu.sync_copy(agg_sh.at[pl.ds(sid * ROWS_PER_TILE, ROWS_PER_TILE)],
                    out_hbm.at[cid, pl.ds(sid * ROWS_PER_TILE, ROWS_PER_TILE)])


@functools.cache
def _make_sc_kernels():
    mesh = plsc.VectorSubcoreMesh(core_axis_name="c", subcore_axis_name="s")
    params = pltpu.CompilerParams(use_tc_tiling_on_sc=False)
    gather = pl.kernel(
        _sc_gather_body,
        out_type=jax.ShapeDtypeStruct((E, 32), jnp.float32),
        mesh=mesh,
        compiler_params=params,
        scratch_types=(
            [pltpu.VMEM((NCHUNK, CHUNK), jnp.int32)]
            + [pltpu.VMEM((CHUNK, 32), jnp.float32)] * NBUF
            + [pltpu.SemaphoreType.DMA] * (2 * NBUF)
        ),
    )
    scatter = pl.kernel(
        _sc_scatter_body,
        out_type=jax.ShapeDtypeStruct((2, N, 32), jnp.float32),
        mesh=mesh,
        compiler_params=params,
        scratch_types=[
            pltpu.VMEM((NCHUNK, CHUNK), jnp.int32),
            pltpu.VMEM((CHUNK, 32), jnp.float32),
            pltpu.VMEM((CHUNK, 32), jnp.float32),
            pltpu.VMEM_SHARED((N, 32), jnp.float32),
            pltpu.SemaphoreType.DMA,
            pltpu.SemaphoreType.DMA,
        ],
    )
    return gather, scatter


def _tp_body(ev_ref, elen_ref, xg_ref, W1_ref, b1_ref, W2_ref, b2_ref,
             Rtile_ref, Rrep_ref, R112_ref, BigS_ref, Rsv_ref, Ryk_ref,
             mask_ref, out_ref):
    f32 = jnp.float32
    dn_t = (((0,), (0,)), ((), ()))
    evT = ev_ref[...]
    lenT = elen_ref[...]
    xg = xg_ref[...]
    r2 = jnp.sum(evT * evT, axis=0, keepdims=True)
    inv = np.float32(np.sqrt(3.0)) / jnp.maximum(jnp.sqrt(r2), 1e-12)
    y1eT = jnp.concatenate([jnp.ones_like(lenT), evT * inv], axis=0)
    pre = lax.dot_general(lenT, W1_ref[...], dn_t,
                          preferred_element_type=f32) + b1_ref[...]
    hid = pre * jax.nn.sigmoid(pre)
    wexp = (jnp.dot(hid.astype(jnp.bfloat16), W2_ref[...],
                    preferred_element_type=f32) + b2_ref[...])
    xgrep = jnp.dot(xg, Rtile_ref[...], preferred_element_type=f32)
    Y = lax.dot_general(y1eT, Rrep_ref[...], dn_t,
                        preferred_element_type=f32)
    F = xgrep * Y
    Lexp = jnp.dot(F.astype(jnp.bfloat16), R112_ref[...],
                   preferred_element_type=f32)
    OUT = jnp.dot((Lexp * wexp).astype(jnp.bfloat16), BigS_ref[...],
                  preferred_element_type=f32)
    souter = (jnp.dot(OUT, Rsv_ref[...], preferred_element_type=f32)
              * lax.dot_general(y1eT, Ryk_ref[...], dn_t,
                                preferred_element_type=f32))
    out_ref[...] = OUT * mask_ref[...] + souter


def _gate_body(h_ref, part_ref, Wg_ref, bg_ref, out_ref):
    h = h_ref[...]
    agg = part_ref[0] + part_ref[1]
    gate = jax.nn.sigmoid(
        jnp.dot(h[:, :MUL0], Wg_ref[...], preferred_element_type=jnp.float32)
        + bg_ref[...])
    out_ref[...] = h + jnp.concatenate(
        [agg[:, :MUL0], agg[:, MUL0:DIM] * gate], axis=1)


def _full(shape):
    return pl.BlockSpec(shape, lambda i: tuple(0 for _ in shape))


@jax.jit
def kernel(h, edge_index, edge_vec, edge_len, W1, b1, W2, b2, Wg, bg):
    sender = edge_index[0]
    receiver = edge_index[1]
    sp = sender.reshape(NW, NCHUNK, CHUNK)
    rp = receiver.reshape(NW, NCHUNK, CHUNK)
    h32 = jnp.pad(h, ((0, 0), (0, 32 - DIM)))
    evT = edge_vec.T
    elT = edge_len.reshape(1, E)
    zero_init = jnp.zeros((N, 32), jnp.float32)
    wsc = jnp.asarray(_wscale)
    W2e = jnp.concatenate([W2[:, :384], W2[:, 384:400], W2[:, 384:400],
                           W2[:, 384:400], W2[:, 400:416], W2[:, 400:416],
                           W2[:, 400:416]], axis=1) * wsc
    b2e = jnp.concatenate([b2[:384], b2[384:400], b2[384:400], b2[384:400],
                           b2[400:416], b2[400:416], b2[400:416]]) * wsc
    W2eb = W2e.astype(jnp.bfloat16)

    sc_gather, sc_scatter = _make_sc_kernels()
    xg = sc_gather(h32, sp)

    grid = E // BLK_E
    msg = pl.pallas_call(
        _tp_body,
        grid=(grid,),
        in_specs=[
            pl.BlockSpec((3, BLK_E), lambda i: (0, i)),
            pl.BlockSpec((1, BLK_E), lambda i: (0, i)),
            pl.BlockSpec((BLK_E, 32), lambda i: (i, 0)),
            _full((1, RMH)), _full((1, RMH)), _full((RMH, WEXP)),
            _full((1, WEXP)), _full((32, FDIM)), _full((4, FDIM)),
            _full((FDIM, WEXP)), _full((WEXP, 32)), _full((32, 32)),
            _full((4, 32)), _full((1, 32)),
        ],
        out_specs=pl.BlockSpec((BLK_E, 32), lambda i: (i, 0)),
        out_shape=jax.ShapeDtypeStruct((E, 32), jnp.float32),
    )(evT, elT, xg,
      W1, b1.reshape(1, RMH), W2eb, b2e.reshape(1, WEXP),
      jnp.asarray(_Rtile), jnp.asarray(_Rrep),
      jnp.asarray(_R112).astype(jnp.bfloat16),
      jnp.asarray(_BigS).astype(jnp.bfloat16),
      jnp.asarray(_Rsv32), jnp.asarray(_Ryk32),
      jnp.asarray(_mask32))

    parts = sc_scatter(msg, rp, zero_init)

    out = pl.pallas_call(
        _gate_body,
        grid=(1,),
        in_specs=[
            _full((N, DIM)), _full((2, N, 32)),
            _full((MUL0, DIM - MUL0)), _full((1, DIM - MUL0)),
        ],
        out_specs=_full((N, DIM)),
        out_shape=jax.ShapeDtypeStruct((N, DIM), jnp.float32),
    )(h, parts, Wg, bg.reshape(1, DIM - MUL0))
    return out


# speedup vs baseline: 1.3792x; 1.0063x over previous
import functools

import jax
import jax.numpy as jnp
import numpy as np
from jax import lax
from jax.experimental import pallas as pl
from jax.experimental.pallas import tpu as pltpu
from jax.experimental.pallas import tpu_sc as plsc

N = 10000
E = 160000
MUL0 = 16
MUL1 = 4
DIM = MUL0 + 3 * MUL1
RMH = 64
WEXP = 480
LDIM = 44
ODIM = 44

NW = 32
CHUNK = 125
NCHUNK = 40
ROWS_PER_TILE = N // 16

BLK_E = 3200


FDIM = 112


def _build_consts():
    c_w1 = 1.0 / (4.0 * np.sqrt(2.0))
    c_w2 = 1.0 / (np.sqrt(3.0) * 2.0 * np.sqrt(2.0))
    c_w3 = 1.0 / (4.0 * np.sqrt(3.0))
    c_w4 = 1.0 / (2.0 * np.sqrt(3.0))
    c_w5 = 1.0 / (2.0 * np.sqrt(6.0))
    Rtile = np.zeros((32, FDIM), np.float32)
    Rrep = np.zeros((4, FDIM), np.float32)
    for kp in range(4):
        for c in range(DIM):
            Rtile[c, 28 * kp + c] = 1.0
            Rrep[kp, 28 * kp + c] = 1.0
    R112 = np.zeros((FDIM, WEXP), np.float32)
    BigS = np.zeros((WEXP, 32), np.float32)
    wscale = np.zeros((WEXP,), np.float32)
    wscale[0:256] = c_w1
    wscale[256:320] = c_w2
    wscale[320:384] = c_w3
    wscale[384:432] = c_w4
    wscale[432:480] = c_w5
    for i in range(16):
        for o in range(16):
            R112[i, 16 * i + o] = 1.0
            BigS[16 * i + o, o] = 1.0
        for v in range(4):
            R112[i, 320 + 4 * i + v] = 1.0
            BigS[320 + 4 * i + v, 28 + v] = 1.0
    for u in range(4):
        for o in range(16):
            for k in range(3):
                R112[28 * (k + 1) + 16 + 3 * u + k, 256 + 16 * u + o] = 1.0
            BigS[256 + 16 * u + o, o] = 1.0
    for k in range(3):
        a, b = (k + 1) % 3, (k + 2) % 3
        for u in range(4):
            for v in range(4):
                R112[16 + 3 * u + k, 384 + 16 * k + 4 * u + v] = 1.0
                BigS[384 + 16 * k + 4 * u + v, 16 + 3 * v + k] = 1.0
                R112[28 * (b + 1) + 16 + 3 * u + a, 432 + 16 * k + 4 * u + v] = 1.0
                R112[28 * (a + 1) + 16 + 3 * u + b, 432 + 16 * k + 4 * u + v] = -1.0
                BigS[432 + 16 * k + 4 * u + v, 16 + 3 * v + k] = 1.0
    Rsv32 = np.zeros((32, 32), np.float32)
    Ryk32 = np.zeros((4, 32), np.float32)
    for v in range(4):
        for k in range(3):
            Rsv32[28 + v, 16 + 3 * v + k] = 1.0
            Ryk32[1 + k, 16 + 3 * v + k] = 1.0
    mask32 = np.ones((1, 32), np.float32)
    mask32[0, 28:] = 0.0
    return Rtile, Rrep, R112, BigS, Rsv32, Ryk32, mask32, wscale


(_Rtile, _Rrep, _R112, _BigS, _Rsv32, _Ryk32, _mask32,
 _wscale) = _build_consts()

NBUF = 4


def _sc_gather_body(h_hbm, idx_hbm, out_hbm, idx_v,
                    buf0, buf1, buf2, buf3,
                    gs0, gs1, gs2, gs3, ws0, ws1, ws2, ws3):
    wid = lax.axis_index("s") * 2 + lax.axis_index("c")
    base = wid * (NCHUNK * CHUNK)
    pltpu.sync_copy(idx_hbm.at[wid], idx_v)
    bufs = (buf0, buf1, buf2, buf3)
    gsems = (gs0, gs1, gs2, gs3)
    wsems = (ws0, ws1, ws2, ws3)
    for b in range(NBUF):
        pltpu.async_copy(h_hbm.at[idx_v.at[b]], bufs[b], gsems[b])

    def body(jo, _):
        for b in range(NBUF):
            j = jo * NBUF + b
            pltpu.make_async_copy(
                h_hbm.at[idx_v.at[j]], bufs[b], gsems[b]).wait()
            pltpu.async_copy(
                bufs[b], out_hbm.at[pl.ds(base + j * CHUNK, CHUNK)], wsems[b])

            @pl.when(j + NBUF < NCHUNK)
            def _():
                pltpu.make_async_copy(
                    bufs[b], out_hbm.at[pl.ds(base, CHUNK)], wsems[b]).wait()
                pltpu.async_copy(h_hbm.at[idx_v.at[j + NBUF]], bufs[b],
                                 gsems[b])
        return 0

    lax.fori_loop(0, NCHUNK // NBUF, body, 0)
    for b in range(NBUF):
        pltpu.make_async_copy(
            bufs[b], out_hbm.at[pl.ds(base, CHUNK)], wsems[b]).wait()


def _sc_scatter_body(msg_hbm, idx_hbm, zero_hbm, out_hbm,
                     idx_v, buf0, buf1, agg_sh, sem0, sem1):
    cid = lax.axis_index("c")
    sid = lax.axis_index("s")
    wid = sid * 2 + cid
    base = wid * (NCHUNK * CHUNK)
    pltpu.sync_copy(zero_hbm.at[pl.ds(sid * ROWS_PER_TILE, ROWS_PER_TILE)],
                    agg_sh.at[pl.ds(sid * ROWS_PER_TILE, ROWS_PER_TILE)])
    pltpu.sync_copy(idx_hbm.at[wid], idx_v)
    plsc.subcore_barrier()
    bufs = (buf0, buf1)
    sems = (sem0, sem1)
    pltpu.async_copy(msg_hbm.at[pl.ds(base, CHUNK)], buf0, sem0)

    def body(j, _):
        slot = lax.rem(j, 2)

        @pl.when(j + 1 < NCHUNK)
        def _():
            for s in range(2):
                @pl.when(slot != s)
                def _():
                    pltpu.async_copy(
                        msg_hbm.at[pl.ds(base + (j + 1) * CHUNK, CHUNK)],
                        bufs[s], sems[s])

        for s in range(2):
            @pl.when(slot == s)
            def _():
                pltpu.make_async_copy(
                    msg_hbm.at[pl.ds(base + j * CHUNK, CHUNK)],
                    bufs[s], sems[s]).wait()
                pltpu.sync_copy(bufs[s], agg_sh.at[idx_v.at[j]], add=True)
        return 0

    lax.fori_loop(0, NCHUNK, body, 0)
    plsc.subcore_barrier()
    pltpu.sync_copy(agg_sh.at[pl.ds(sid * ROWS_PER_TILE, ROWS_PER_TILE)],
                    out_hbm.at[cid, pl.ds(sid * ROWS_PER_TILE, ROWS_PER_TILE)])


@functools.cache
def _make_sc_kernels():
    mesh = plsc.VectorSubcoreMesh(core_axis_name="c", subcore_axis_name="s")
    params = pltpu.CompilerParams(use_tc_tiling_on_sc=False)
    gather = pl.kernel(
        _sc_gather_body,
        out_type=jax.ShapeDtypeStruct((E, 32), jnp.float32),
        mesh=mesh,
        compiler_params=params,
        scratch_types=(
            [pltpu.VMEM((NCHUNK, CHUNK), jnp.int32)]
            + [pltpu.VMEM((CHUNK, 32), jnp.float32)] * NBUF
            + [pltpu.SemaphoreType.DMA] * (2 * NBUF)
        ),
    )
    scatter = pl.kernel(
        _sc_scatter_body,
        out_type=jax.ShapeDtypeStruct((2, N, 32), jnp.float32),
        mesh=mesh,
        compiler_params=params,
        scratch_types=[
            pltpu.VMEM((NCHUNK, CHUNK), jnp.int32),
            pltpu.VMEM((CHUNK, 32), jnp.float32),
            pltpu.VMEM((CHUNK, 32), jnp.float32),
            pltpu.VMEM_SHARED((N, 32), jnp.float32),
            pltpu.SemaphoreType.DMA,
            pltpu.SemaphoreType.DMA,
        ],
    )
    return gather, scatter


def _tp_body(ev_ref, elen_ref, xg_ref, W1_ref, b1_ref, W2_ref,
             Rtile_ref, Rrep_ref, R112_ref, BigS_ref, Rsv_ref, Ryk_ref,
             mask_ref, out_ref):
    f32 = jnp.float32
    dn_t = (((0,), (0,)), ((), ()))
    evT = ev_ref[...]
    lenT = elen_ref[...]
    xg = xg_ref[...]
    r2 = jnp.sum(evT * evT, axis=0, keepdims=True)
    inv = np.float32(np.sqrt(3.0)) / jnp.maximum(jnp.sqrt(r2), 1e-12)
    y1eT = jnp.concatenate([jnp.ones_like(lenT), evT * inv], axis=0)
    bf16 = jnp.bfloat16
    pre = lax.dot_general(lenT, W1_ref[...], dn_t,
                          preferred_element_type=f32) + b1_ref[...]
    hid = pre * jax.nn.sigmoid(pre)
    wexp = jnp.dot(hid.astype(bf16), W2_ref[...],
                   preferred_element_type=f32)
    xgrep = jnp.dot(xg, Rtile_ref[...], preferred_element_type=f32)
    Y = lax.dot_general(y1eT, Rrep_ref[...], dn_t,
                        preferred_element_type=f32)
    F = xgrep * Y
    Lexp = jnp.dot(F.astype(bf16), R112_ref[...],
                   preferred_element_type=f32)
    OUT = jnp.dot((Lexp * wexp).astype(bf16), BigS_ref[...],
                  preferred_element_type=f32)
    souter = (jnp.dot(OUT, Rsv_ref[...], preferred_element_type=f32)
              * lax.dot_general(y1eT, Ryk_ref[...], dn_t,
                                preferred_element_type=f32))
    out_ref[...] = OUT * mask_ref[...] + souter


def _gate_body(h_ref, part_ref, Wg_ref, bg_ref, out_ref):
    h = h_ref[...]
    agg = part_ref[0] + part_ref[1]
    gate = jax.nn.sigmoid(
        jnp.dot(h[:, :MUL0], Wg_ref[...], preferred_element_type=jnp.float32)
        + bg_ref[...])
    out_ref[...] = h + jnp.concatenate(
        [agg[:, :MUL0], agg[:, MUL0:DIM] * gate], axis=1)


def _full(shape):
    return pl.BlockSpec(shape, lambda i: tuple(0 for _ in shape))


@jax.jit
def kernel(h, edge_index, edge_vec, edge_len, W1, b1, W2, b2, Wg, bg):
    sender = edge_index[0]
    receiver = edge_index[1]
    sp = sender.reshape(NW, NCHUNK, CHUNK)
    rp = receiver.reshape(NW, NCHUNK, CHUNK)
    h32 = jnp.pad(h, ((0, 0), (0, 32 - DIM)))
    evT = edge_vec.T
    elT = edge_len.reshape(1, E)
    zero_init = jnp.zeros((N, 32), jnp.float32)
    wsc = jnp.asarray(_wscale)
    W2e = jnp.concatenate([W2[:, :384], W2[:, 384:400], W2[:, 384:400],
                           W2[:, 384:400], W2[:, 400:416], W2[:, 400:416],
                           W2[:, 400:416]], axis=1) * wsc
    b2e = jnp.concatenate([b2[:384], b2[384:400], b2[384:400], b2[384:400],
                           b2[400:416], b2[400:416], b2[400:416]]) * wsc
    silu20 = 20.0 / (1.0 + np.exp(-20.0))
    W2eb = jnp.concatenate(
        [W2e, b2e.reshape(1, WEXP) / silu20,
         jnp.zeros((7, WEXP), jnp.float32)], axis=0).astype(jnp.bfloat16)
    W1e = jnp.concatenate([W1, jnp.zeros((1, 8), jnp.float32)], axis=1)
    b1e = jnp.concatenate(
        [b1, jnp.full((8,), 20.0, jnp.float32)]).reshape(1, RMH + 8)

    sc_gather, sc_scatter = _make_sc_kernels()
    xg = sc_gather(h32, sp)

    grid = E // BLK_E
    msg = pl.pallas_call(
        _tp_body,
        grid=(grid,),
        in_specs=[
            pl.BlockSpec((3, BLK_E), lambda i: (0, i)),
            pl.BlockSpec((1, BLK_E), lambda i: (0, i)),
            pl.BlockSpec((BLK_E, 32), lambda i: (i, 0)),
            _full((1, RMH + 8)), _full((1, RMH + 8)),
            _full((RMH + 8, WEXP)), _full((32, FDIM)), _full((4, FDIM)),
            _full((FDIM, WEXP)), _full((WEXP, 32)), _full((32, 32)),
            _full((4, 32)), _full((1, 32)),
        ],
        out_specs=pl.BlockSpec((BLK_E, 32), lambda i: (i, 0)),
        out_shape=jax.ShapeDtypeStruct((E, 32), jnp.float32),
    )(evT, elT, xg,
      W1e, b1e, W2eb,
      jnp.asarray(_Rtile), jnp.asarray(_Rrep),
      jnp.asarray(_R112).astype(jnp.bfloat16),
      jnp.asarray(_BigS).astype(jnp.bfloat16),
      jnp.asarray(_Rsv32), jnp.asarray(_Ryk32),
      jnp.asarray(_mask32))

    parts = sc_scatter(msg, rp, zero_init)

    out = pl.pallas_call(
        _gate_body,
        grid=(1,),
        in_specs=[
            _full((N, DIM)), _full((2, N, 32)),
            _full((MUL0, DIM - MUL0)), _full((1, DIM - MUL0)),
        ],
        out_specs=_full((N, DIM)),
        out_shape=jax.ShapeDtypeStruct((N, DIM), jnp.float32),
    )(h, parts, Wg, bg.reshape(1, DIM - MUL0))
    return out
